# baseline (speedup 1.0000x reference)
import functools

import jax
import jax.numpy as jnp
from jax import lax
from jax.experimental import pallas as pl
from jax.experimental.pallas import tpu as pltpu

B, N, M = 2, 1536, 50
NB = B * N
H_INIT, H_A, H_B, H_G, N_CONV = 92, 64, 32, 32, 4
N_TYPES = 100
D_MODEL, NHEAD, D_FF, E_DIM = 128, 8, 128, 32
HD = D_MODEL // NHEAD
L = 512
AB = 1024
NBLK = NB // AB
EPS = 1e-5

_INTERPRET = False

SC_NC, SC_NS = 2, 16
NW = SC_NC * SC_NS
ROWS = M * NB
RPW = ROWS // NW
CH = 120
NCH = RPW // CH


def _sc_gather(bv, idx3):
    from jax.experimental.pallas import tpu_sc as plsc
    mesh = plsc.VectorSubcoreMesh(core_axis_name="c", subcore_axis_name="s",
                                  num_cores=SC_NC)

    D = 3
    NBUF = 2 * D

    @functools.partial(
        pl.kernel,
        out_type=jax.ShapeDtypeStruct((ROWS, H_A), jnp.float32),
        mesh=mesh,
        compiler_params=pltpu.CompilerParams(use_tc_tiling_on_sc=False),
        scratch_types=[
            pltpu.VMEM((NCH, CH), jnp.int32),
            pltpu.VMEM((NBUF, CH, H_A), jnp.float32),
            pltpu.SemaphoreType.DMA,
            pltpu.SemaphoreType.DMA,
        ],
    )
    def k(table_hbm, idx_hbm, out_hbm, idx_v, bufs, sem_g, sem_o):
        wid = lax.axis_index("s") * SC_NC + lax.axis_index("c")
        base = wid * RPW
        pltpu.sync_copy(idx_hbm.at[wid], idx_v)
        gcps = [None] * NCH
        ocps = [None] * NCH

        def start_gather(j):
            gcps[j] = pltpu.async_copy(
                table_hbm.at[idx_v.at[j]], bufs.at[j % NBUF], sem_g)

        for j in range(D):
            start_gather(j)
        for j in range(NCH):
            pre = j + D
            if pre < NCH:
                if pre >= NBUF:
                    ocps[pre - NBUF].wait()
                start_gather(pre)
            gcps[j].wait()
            ocps[j] = pltpu.async_copy(
                bufs.at[j % NBUF], out_hbm.at[pl.ds(base + j * CH, CH)],
                sem_o)
        for j in range(NCH - NBUF, NCH):
            ocps[j].wait()

    return k(bv, idx3)


def _embed_body(idx_ref, table_ref, wemb_ref, bemb_ref, out_ref):
    tw = jnp.dot(table_ref[:], wemb_ref[:], preferred_element_type=jnp.float32)
    oh = (idx_ref[:] == lax.broadcasted_iota(jnp.int32, (NB, N_TYPES), 1))
    oh = oh.astype(jnp.float32)
    out_ref[:] = jnp.dot(oh, tw, preferred_element_type=jnp.float32) + bemb_ref[:]


def _embed(idx, atom_table, wemb, bemb):
    return pl.pallas_call(
        _embed_body,
        out_shape=jax.ShapeDtypeStruct((NB, H_A), jnp.float32),
        interpret=_INTERPRET,
    )(idx.reshape(NB, 1), atom_table, wemb, bemb.reshape(1, H_A))


MB2 = 2
NMB = M // MB2


def _conv_body(g_ref, nb_ref, atom_ref, wf_ref, bf_ref, g1_ref, b1_ref,
               g2_ref, b2_ref, out_ref, gst_ref, nsum_ref, acc_ref):
    s = pl.program_id(0)

    @pl.when(s == 0)
    def _init():
        acc_ref[:] = jnp.zeros_like(acc_ref)

    @pl.when(s < NMB)
    def _phase0():
        atom2 = jnp.broadcast_to(atom_ref[:][None], (MB2, NB, H_A))
        x = jnp.concatenate([atom2.reshape(MB2 * NB, H_A),
                             g_ref[:].reshape(MB2 * NB, H_A),
                             nb_ref[:].reshape(MB2 * NB, H_B)], axis=1)
        gated = (jnp.dot(x, wf_ref[:], preferred_element_type=jnp.float32)
                 + bf_ref[:])
        acc_ref[0:1, :] += jnp.sum(gated, axis=0, keepdims=True)
        acc_ref[1:2, :] += jnp.sum(gated * gated, axis=0, keepdims=True)
        gst_ref[pl.ds(s * MB2, MB2)] = (
            gated.reshape(MB2, NB, D_MODEL).astype(jnp.bfloat16))

    @pl.when(s == NMB)
    def _fin():
        cnt = float(NB * M)
        mu = acc_ref[0:1, :] / cnt
        var = acc_ref[1:2, :] / cnt - mu * mu
        inv = lax.rsqrt(var + EPS)
        scale = inv * g1_ref[:]
        shift = b1_ref[:] - mu * scale
        nsum_ref[:] = jnp.zeros_like(nsum_ref)

        def mstep(m, carry):
            y = gst_ref[pl.ds(m, 1)][0].astype(jnp.float32) * scale + shift
            filt = jax.nn.sigmoid(y[:, :H_A])
            core = jnp.maximum(y[:, H_A:], 0.0)
            nsum_ref[:] += filt * core
            return carry

        lax.fori_loop(0, M, mstep, 0)
        ns = nsum_ref[:]
        mu2 = jnp.sum(ns, axis=0, keepdims=True) / float(NB)
        var2 = jnp.sum(ns * ns, axis=0, keepdims=True) / float(NB) - mu2 * mu2
        inv2 = lax.rsqrt(var2 + EPS)
        bn = (ns - mu2) * (inv2 * g2_ref[:]) + b2_ref[:]
        out_ref[:] = jnp.maximum(atom_ref[:] + bn, 0.0)


def _conv_layer(g, nbr_t, atom_emb, wf, bf, g1, b1, g2, b2):
    cap = NMB - 1
    return pl.pallas_call(
        _conv_body,
        grid=(NMB + 1,),
        in_specs=[
            pl.BlockSpec((MB2, NB, H_A),
                         lambda s: (jnp.minimum(s, cap), 0, 0)),
            pl.BlockSpec((MB2, NB, H_B),
                         lambda s: (jnp.minimum(s, cap), 0, 0)),
            pl.BlockSpec((NB, H_A), lambda s: (0, 0)),
            pl.BlockSpec((2 * H_A + H_B, 2 * H_A), lambda s: (0, 0)),
            pl.BlockSpec((1, 2 * H_A), lambda s: (0, 0)),
            pl.BlockSpec((1, 2 * H_A), lambda s: (0, 0)),
            pl.BlockSpec((1, 2 * H_A), lambda s: (0, 0)),
            pl.BlockSpec((1, H_A), lambda s: (0, 0)),
            pl.BlockSpec((1, H_A), lambda s: (0, 0)),
        ],
        out_specs=pl.BlockSpec((NB, H_A), lambda s: (0, 0)),
        out_shape=jax.ShapeDtypeStruct((NB, H_A), jnp.float32),
        scratch_shapes=[
            pltpu.VMEM((M, NB, D_MODEL), jnp.bfloat16),
            pltpu.VMEM((NB, H_A), jnp.float32),
            pltpu.VMEM((2, D_MODEL), jnp.float32),
        ],
        interpret=_INTERPRET,
    )(g, nbr_t, atom_emb, wf, bf.reshape(1, -1), g1.reshape(1, -1),
      b1.reshape(1, -1), g2.reshape(1, -1), b2.reshape(1, -1))


def _ln_rows(x, g, b):
    mu = jnp.mean(x, axis=-1, keepdims=True)
    d = x - mu
    var = jnp.mean(d * d, axis=-1, keepdims=True)
    return d * lax.rsqrt(var + EPS) * g + b


def _tr_body(res_ref, eps_ref,
             wmu_ref, bmu_ref, wvar_ref, bvar_ref, wfc_ref, bfc_ref,
             wq_ref, bq_ref, wk_ref, bk_ref, wv_ref, bv_ref,
             wo_ref, bo_ref, wff1_ref, bff1_ref, wff2_ref, bff2_ref,
             ln1g_ref, ln1b_ref, ln2g_ref, ln2b_ref,
             wout_ref, bout_ref,
             out_ref, mu_ref, lv_ref):
    scl = 1.0 / jnp.sqrt(float(D_MODEL))
    for b in range(B):
        h = jnp.maximum(res_ref[b], 0.0)
        mu = jnp.dot(h, wmu_ref[:], preferred_element_type=jnp.float32) + bmu_ref[:]
        lv = jnp.dot(h, wvar_ref[:], preferred_element_type=jnp.float32) + bvar_ref[:]
        mu_ref[b] = mu
        lv_ref[b] = lv
        amino = mu + eps_ref[b] * jnp.exp(0.5 * lv)
        x = jnp.dot(amino, wfc_ref[:], preferred_element_type=jnp.float32) + bfc_ref[:]
        for i in range(N_CONV):
            attn_out = jnp.zeros((L, E_DIM), jnp.float32)
            for hh in range(NHEAD):
                qh = jnp.dot(x, wq_ref[i, hh],
                             preferred_element_type=jnp.float32) + bq_ref[i, hh]
                kh = jnp.dot(x, wk_ref[i, hh],
                             preferred_element_type=jnp.float32) + bk_ref[i, hh]
                vh = jnp.dot(x, wv_ref[i, hh],
                             preferred_element_type=jnp.float32) + bv_ref[i, hh]
                s = lax.dot_general(qh, kh, (((1,), (1,)), ((), ())),
                                    preferred_element_type=jnp.float32) * scl
                s = s - jnp.max(s, axis=-1, keepdims=True)
                e = jnp.exp(s)
                sm = e / jnp.sum(e, axis=-1, keepdims=True)
                ctx = jnp.dot(sm, vh, preferred_element_type=jnp.float32)
                attn_out += jnp.dot(ctx, wo_ref[i, hh],
                                    preferred_element_type=jnp.float32)
            x = _ln_rows(x + attn_out + bo_ref[i], ln1g_ref[i], ln1b_ref[i])
            ff = jnp.maximum(
                jnp.dot(x, wff1_ref[i], preferred_element_type=jnp.float32)
                + bff1_ref[i], 0.0)
            ff = jnp.dot(ff, wff2_ref[i], preferred_element_type=jnp.float32) \
                + bff2_ref[i]
            x = _ln_rows(x + ff, ln2g_ref[i], ln2b_ref[i])
        out_ref[b] = jnp.dot(x, wout_ref[:], preferred_element_type=jnp.float32) \
            + bout_ref[:]


def _transformer(res, eps, p):
    wq = p["tWq"].reshape(N_CONV, E_DIM, NHEAD, HD).transpose(0, 2, 1, 3)
    wk = p["tWk"].reshape(N_CONV, E_DIM, NHEAD, HD).transpose(0, 2, 1, 3)
    wv = p["tWv"].reshape(N_CONV, E_DIM, NHEAD, HD).transpose(0, 2, 1, 3)
    bq = p["tbq"].reshape(N_CONV, NHEAD, 1, HD)
    bk = p["tbk"].reshape(N_CONV, NHEAD, 1, HD)
    bv = p["tbv"].reshape(N_CONV, NHEAD, 1, HD)
    wo = p["tWo"].reshape(N_CONV, NHEAD, HD, E_DIM)
    wout = jnp.zeros((E_DIM, 16), jnp.float32).at[:, :9].set(p["Wout"])
    bout = jnp.zeros((1, 16), jnp.float32).at[:, :9].set(p["bout"])
    r2 = lambda a: a.reshape(N_CONV, 1, -1)
    out, mu, lv = pl.pallas_call(
        _tr_body,
        out_shape=[
            jax.ShapeDtypeStruct((B, L, 16), jnp.float32),
            jax.ShapeDtypeStruct((B, L, H_G), jnp.float32),
            jax.ShapeDtypeStruct((B, L, H_G), jnp.float32),
        ],
        interpret=_INTERPRET,
    )(res, eps,
      p["Wmu"], p["bmu"].reshape(1, -1), p["Wvar"], p["bvar"].reshape(1, -1),
      p["Wfc"], p["bfc"].reshape(1, -1),
      wq, bq, wk, bk, wv, bv, wo, r2(p["tbo"]),
      p["tWff1"], r2(p["tbff1"]), p["tWff2"], r2(p["tbff2"]),
      r2(p["tln1g"]), r2(p["tln1b"]), r2(p["tln2g"]), r2(p["tln2b"]),
      wout, bout)
    return out[:, :, :9], mu, lv


def kernel(atom_emb_idx, nbr_emb, nbr_adj_list, atom_table, params):
    p = params
    idx = atom_emb_idx.reshape(NB).astype(jnp.int32)
    adj = (nbr_adj_list.astype(jnp.int32)
           + (jnp.arange(B, dtype=jnp.int32) * N)[:, None, None])
    idx_t = adj.transpose(2, 0, 1).reshape(M, NB)
    idx3 = idx_t.reshape(NW, NCH, CH)
    nbr_t = nbr_emb.transpose(2, 0, 1, 3).reshape(M, NB, H_B)

    atom_emb = _embed(idx, atom_table, p["W_embed"], p["b_embed"])
    for i in range(N_CONV):
        g = _sc_gather(atom_emb, idx3).reshape(M, NB, H_A)
        atom_emb = _conv_layer(g, nbr_t, atom_emb, p["conv_Wf"][i],
                               p["conv_bf"][i],
                               p["conv_g1"][i], p["conv_b1"][i],
                               p["conv_g2"][i], p["conv_b2"][i])

    res = atom_emb.reshape(B, L, 3 * H_A)
    eps = jax.random.normal(jax.random.key(1234), (B, L, H_G), dtype=jnp.float32)
    out, mu, lv = _transformer(res, eps, p)
    return out.reshape(B, L, 3, 3), mu, lv


# speedup vs baseline: 1.2524x; 1.2524x over previous
import functools

import jax
import jax.numpy as jnp
from jax import lax
from jax.experimental import pallas as pl
from jax.experimental.pallas import tpu as pltpu

B, N, M = 2, 1536, 50
NB = B * N
H_INIT, H_A, H_B, H_G, N_CONV = 92, 64, 32, 32, 4
N_TYPES = 100
D_MODEL, NHEAD, D_FF, E_DIM = 128, 8, 128, 32
HD = D_MODEL // NHEAD
L = 512
AB = 1024
NBLK = NB // AB
EPS = 1e-5

SC_NC, SC_NS = 2, 16
NW = SC_NC * SC_NS
ROWS = M * NB
RPW = ROWS // NW
CH = 120
NCH = RPW // CH


def _sc_gather(bv, idx3):
    from jax.experimental.pallas import tpu_sc as plsc
    mesh = plsc.VectorSubcoreMesh(core_axis_name="c", subcore_axis_name="s",
                                  num_cores=SC_NC)

    D = 3
    NBUF = 2 * D

    @functools.partial(
        pl.kernel,
        out_type=jax.ShapeDtypeStruct((ROWS, D_MODEL), jnp.float32),
        mesh=mesh,
        scratch_types=[
            pltpu.VMEM((NCH, CH), jnp.int32),
            pltpu.VMEM((NBUF, CH, D_MODEL), jnp.float32),
            pltpu.SemaphoreType.DMA,
            pltpu.SemaphoreType.DMA,
        ],
    )
    def k(table_hbm, idx_hbm, out_hbm, idx_v, bufs, sem_g, sem_o):
        wid = lax.axis_index("s") * SC_NC + lax.axis_index("c")
        base = wid * RPW
        pltpu.sync_copy(idx_hbm.at[wid], idx_v)
        gcps = [None] * NCH
        ocps = [None] * NCH

        def start_gather(j):
            gcps[j] = pltpu.async_copy(
                table_hbm.at[idx_v.at[j]], bufs.at[j % NBUF], sem_g)

        for j in range(D):
            start_gather(j)
        for j in range(NCH):
            pre = j + D
            if pre < NCH:
                if pre >= NBUF:
                    ocps[pre - NBUF].wait()
                start_gather(pre)
            gcps[j].wait()
            ocps[j] = pltpu.async_copy(
                bufs.at[j % NBUF], out_hbm.at[pl.ds(base + j * CH, CH)],
                sem_o)
        for j in range(NCH - NBUF, NCH):
            ocps[j].wait()

    return k(bv, idx3)


def _embed_body(idx_ref, table_ref, wemb_ref, bemb_ref, wf2_ref,
                out_ref, bv_ref):
    tw = jnp.dot(table_ref[:], wemb_ref[:], preferred_element_type=jnp.float32)
    oh = (idx_ref[:] == lax.broadcasted_iota(jnp.int32, (NB, N_TYPES), 1))
    oh = oh.astype(jnp.float32)
    emb = jnp.dot(oh, tw, preferred_element_type=jnp.float32) + bemb_ref[:]
    out_ref[:] = emb
    bv_ref[:] = jnp.dot(emb, wf2_ref[:], preferred_element_type=jnp.float32)


def _embed(idx, atom_table, wemb, bemb, wf2_0):
    return pl.pallas_call(
        _embed_body,
        out_shape=[
            jax.ShapeDtypeStruct((NB, H_A), jnp.float32),
            jax.ShapeDtypeStruct((NB, D_MODEL), jnp.float32),
        ],
    )(idx.reshape(NB, 1), atom_table, wemb, bemb.reshape(1, H_A), wf2_0)


MB2 = 2
NMB = M // MB2


def _conv_body(gbv_ref, nb_ref, atom_ref, wsn_ref, bf_ref, g1_ref, b1_ref,
               g2_ref, b2_ref, wf2n_ref, out_ref, bv_ref,
               gst_ref, nsum_ref, acc_ref):
    s = pl.program_id(0)

    @pl.when(s == 0)
    def _init():
        acc_ref[:] = jnp.zeros_like(acc_ref)

    @pl.when(s < NMB)
    def _phase0():
        atom2 = jnp.broadcast_to(atom_ref[:][None], (MB2, NB, H_A))
        x = jnp.concatenate([atom2.reshape(MB2 * NB, H_A),
                             nb_ref[:].reshape(MB2 * NB, H_B)], axis=1)
        gated = (jnp.dot(x, wsn_ref[:], preferred_element_type=jnp.float32)
                 + bf_ref[:] + gbv_ref[:].reshape(MB2 * NB, D_MODEL))
        acc_ref[0:1, :] += jnp.sum(gated, axis=0, keepdims=True)
        acc_ref[1:2, :] += jnp.sum(gated * gated, axis=0, keepdims=True)
        gst_ref[pl.ds(s * MB2, MB2)] = (
            gated.reshape(MB2, NB, D_MODEL).astype(jnp.bfloat16))

    @pl.when(s == NMB)
    def _fin():
        cnt = float(NB * M)
        mu = acc_ref[0:1, :] / cnt
        var = acc_ref[1:2, :] / cnt - mu * mu
        inv = lax.rsqrt(var + EPS)
        scale = inv * g1_ref[:]
        shift = b1_ref[:] - mu * scale
        nsum_ref[:] = jnp.zeros_like(nsum_ref)

        def mstep(m, carry):
            y = gst_ref[pl.ds(m, 1)][0].astype(jnp.float32) * scale + shift
            filt = jax.nn.sigmoid(y[:, :H_A])
            core = jnp.maximum(y[:, H_A:], 0.0)
            nsum_ref[:] += filt * core
            return carry

        lax.fori_loop(0, M, mstep, 0)
        ns = nsum_ref[:]
        mu2 = jnp.sum(ns, axis=0, keepdims=True) / float(NB)
        var2 = jnp.sum(ns * ns, axis=0, keepdims=True) / float(NB) - mu2 * mu2
        inv2 = lax.rsqrt(var2 + EPS)
        bn = (ns - mu2) * (inv2 * g2_ref[:]) + b2_ref[:]
        new_atom = jnp.maximum(atom_ref[:] + bn, 0.0)
        out_ref[:] = new_atom
        bv_ref[:] = jnp.dot(new_atom, wf2n_ref[:],
                            preferred_element_type=jnp.float32)


def _conv_layer(gbv, nbr_t, atom_emb, wsn, bf, g1, b1, g2, b2, wf2n):
    cap = NMB - 1
    return pl.pallas_call(
        _conv_body,
        grid=(NMB + 1,),
        in_specs=[
            pl.BlockSpec((MB2, NB, D_MODEL),
                         lambda s: (jnp.minimum(s, cap), 0, 0)),
            pl.BlockSpec((MB2, NB, H_B),
                         lambda s: (jnp.minimum(s, cap), 0, 0)),
            pl.BlockSpec((NB, H_A), lambda s: (0, 0)),
            pl.BlockSpec((H_A + H_B, 2 * H_A), lambda s: (0, 0)),
            pl.BlockSpec((1, 2 * H_A), lambda s: (0, 0)),
            pl.BlockSpec((1, 2 * H_A), lambda s: (0, 0)),
            pl.BlockSpec((1, 2 * H_A), lambda s: (0, 0)),
            pl.BlockSpec((1, H_A), lambda s: (0, 0)),
            pl.BlockSpec((1, H_A), lambda s: (0, 0)),
            pl.BlockSpec((H_A, D_MODEL), lambda s: (0, 0)),
        ],
        out_specs=[
            pl.BlockSpec((NB, H_A), lambda s: (0, 0)),
            pl.BlockSpec((NB, D_MODEL), lambda s: (0, 0)),
        ],
        out_shape=[
            jax.ShapeDtypeStruct((NB, H_A), jnp.float32),
            jax.ShapeDtypeStruct((NB, D_MODEL), jnp.float32),
        ],
        scratch_shapes=[
            pltpu.VMEM((M, NB, D_MODEL), jnp.bfloat16),
            pltpu.VMEM((NB, H_A), jnp.float32),
            pltpu.VMEM((2, D_MODEL), jnp.float32),
        ],
    )(gbv, nbr_t, atom_emb, wsn, bf.reshape(1, -1), g1.reshape(1, -1),
      b1.reshape(1, -1), g2.reshape(1, -1), b2.reshape(1, -1), wf2n)


def _ln_rows(x, g, b):
    mu = jnp.mean(x, axis=-1, keepdims=True)
    d = x - mu
    var = jnp.mean(d * d, axis=-1, keepdims=True)
    return d * lax.rsqrt(var + EPS) * g + b


def _tr_body(res_ref, eps_ref,
             wmu_ref, bmu_ref, wvar_ref, bvar_ref, wfc_ref, bfc_ref,
             wq_ref, bq_ref, wk_ref, bk_ref, wv_ref, bv_ref,
             wo_ref, bo_ref, wff1_ref, bff1_ref, wff2_ref, bff2_ref,
             ln1g_ref, ln1b_ref, ln2g_ref, ln2b_ref,
             wout_ref, bout_ref,
             out_ref, mu_ref, lv_ref):
    scl = 1.0 / jnp.sqrt(float(D_MODEL))
    for b in range(B):
        h = jnp.maximum(res_ref[b], 0.0)
        mu = jnp.dot(h, wmu_ref[:], preferred_element_type=jnp.float32) + bmu_ref[:]
        lv = jnp.dot(h, wvar_ref[:], preferred_element_type=jnp.float32) + bvar_ref[:]
        mu_ref[b] = mu
        lv_ref[b] = lv
        amino = mu + eps_ref[b] * jnp.exp(0.5 * lv)
        x = jnp.dot(amino, wfc_ref[:], preferred_element_type=jnp.float32) + bfc_ref[:]
        for i in range(N_CONV):
            attn_out = jnp.zeros((L, E_DIM), jnp.float32)
            for hh in range(NHEAD):
                qh = jnp.dot(x, wq_ref[i, hh],
                             preferred_element_type=jnp.float32) + bq_ref[i, hh]
                kh = jnp.dot(x, wk_ref[i, hh],
                             preferred_element_type=jnp.float32) + bk_ref[i, hh]
                vh = jnp.dot(x, wv_ref[i, hh],
                             preferred_element_type=jnp.float32) + bv_ref[i, hh]
                s = lax.dot_general(qh, kh, (((1,), (1,)), ((), ())),
                                    preferred_element_type=jnp.float32) * scl
                s = s - jnp.max(s, axis=-1, keepdims=True)
                e = jnp.exp(s)
                sm = e / jnp.sum(e, axis=-1, keepdims=True)
                ctx = jnp.dot(sm, vh, preferred_element_type=jnp.float32)
                attn_out += jnp.dot(ctx, wo_ref[i, hh],
                                    preferred_element_type=jnp.float32)
            x = _ln_rows(x + attn_out + bo_ref[i], ln1g_ref[i], ln1b_ref[i])
            ff = jnp.maximum(
                jnp.dot(x, wff1_ref[i], preferred_element_type=jnp.float32)
                + bff1_ref[i], 0.0)
            ff = jnp.dot(ff, wff2_ref[i], preferred_element_type=jnp.float32) \
                + bff2_ref[i]
            x = _ln_rows(x + ff, ln2g_ref[i], ln2b_ref[i])
        out_ref[b] = jnp.dot(x, wout_ref[:], preferred_element_type=jnp.float32) \
            + bout_ref[:]


def _transformer(res, eps, p):
    wq = p["tWq"].reshape(N_CONV, E_DIM, NHEAD, HD).transpose(0, 2, 1, 3)
    wk = p["tWk"].reshape(N_CONV, E_DIM, NHEAD, HD).transpose(0, 2, 1, 3)
    wv = p["tWv"].reshape(N_CONV, E_DIM, NHEAD, HD).transpose(0, 2, 1, 3)
    bq = p["tbq"].reshape(N_CONV, NHEAD, 1, HD)
    bk = p["tbk"].reshape(N_CONV, NHEAD, 1, HD)
    bv = p["tbv"].reshape(N_CONV, NHEAD, 1, HD)
    wo = p["tWo"].reshape(N_CONV, NHEAD, HD, E_DIM)
    wout = jnp.zeros((E_DIM, 16), jnp.float32).at[:, :9].set(p["Wout"])
    bout = jnp.zeros((1, 16), jnp.float32).at[:, :9].set(p["bout"])
    r2 = lambda a: a.reshape(N_CONV, 1, -1)
    out, mu, lv = pl.pallas_call(
        _tr_body,
        out_shape=[
            jax.ShapeDtypeStruct((B, L, 16), jnp.float32),
            jax.ShapeDtypeStruct((B, L, H_G), jnp.float32),
            jax.ShapeDtypeStruct((B, L, H_G), jnp.float32),
        ],
    )(res, eps,
      p["Wmu"], p["bmu"].reshape(1, -1), p["Wvar"], p["bvar"].reshape(1, -1),
      p["Wfc"], p["bfc"].reshape(1, -1),
      wq, bq, wk, bk, wv, bv, wo, r2(p["tbo"]),
      p["tWff1"], r2(p["tbff1"]), p["tWff2"], r2(p["tbff2"]),
      r2(p["tln1g"]), r2(p["tln1b"]), r2(p["tln2g"]), r2(p["tln2b"]),
      wout, bout)
    return out[:, :, :9], mu, lv


def kernel(atom_emb_idx, nbr_emb, nbr_adj_list, atom_table, params):
    p = params
    idx = atom_emb_idx.reshape(NB).astype(jnp.int32)
    adj = (nbr_adj_list.astype(jnp.int32)
           + (jnp.arange(B, dtype=jnp.int32) * N)[:, None, None])
    idx_t = adj.transpose(2, 0, 1).reshape(M, NB)
    idx3 = idx_t.reshape(NW, NCH, CH)
    nbr_t = nbr_emb.transpose(2, 0, 1, 3).reshape(M, NB, H_B)

    wf2 = [p["conv_Wf"][i][H_A:2 * H_A] for i in range(N_CONV)]
    wsn = [jnp.concatenate([p["conv_Wf"][i][:H_A], p["conv_Wf"][i][2 * H_A:]],
                           axis=0) for i in range(N_CONV)]

    atom_emb, bv = _embed(idx, atom_table, p["W_embed"], p["b_embed"], wf2[0])
    for i in range(N_CONV):
        gbv = _sc_gather(bv, idx3).reshape(M, NB, D_MODEL)
        atom_emb, bv = _conv_layer(gbv, nbr_t, atom_emb, wsn[i],
                                   p["conv_bf"][i],
                                   p["conv_g1"][i], p["conv_b1"][i],
                                   p["conv_g2"][i], p["conv_b2"][i],
                                   wf2[(i + 1) % N_CONV])

    res = atom_emb.reshape(B, L, 3 * H_A)
    eps = jax.random.normal(jax.random.key(1234), (B, L, H_G), dtype=jnp.float32)
    out, mu, lv = _transformer(res, eps, p)
    return out.reshape(B, L, 3, 3), mu, lv
